# baseline (speedup 1.0000x reference)
import jax
import jax.numpy as jnp
from jax.experimental import pallas as pl
from jax.experimental.pallas import tpu as pltpu

_L = 4096
_F = 8
_H = 64
_NC = 2
_P = _L // 4
_G = 16
_GH = _G // 2
_GP = _G * _P
_GPH = _GH * _P

_ISQRT2 = 0.7071067811865476
_ISQRT3 = 0.5773502691896258


def _model_block(x_ref, c1_ref, c1b_ref, c2_ref, c2b_ref,
                 g1w_ref, g1b_ref, g2w_ref, g2b_ref,
                 fcw_ref, fcb_ref, out_ref):
    f32 = jnp.float32
    pos = jax.lax.broadcasted_iota(jnp.int32, (1, _GPH), 1) % _P
    first = pos == 0
    last = pos == _P - 1
    dinv = jnp.where(first | last, _ISQRT2, _ISQRT3).astype(f32)

    def agg(u):
        ul = jnp.where(first, 0.0, jnp.roll(u, 1, axis=1))
        ur = jnp.where(last, 0.0, jnp.roll(u, -1, axis=1))
        return ul + u + ur

    def half(xp):
        x3m = jnp.where(first, 0.0, jnp.roll(xp[3:4], 1, axis=1))
        x0p = jnp.where(last, 0.0, jnp.roll(xp[0:1], -1, axis=1))
        x6 = jnp.concatenate([x3m, xp, x0p], axis=0)
        y = jnp.dot(c1_ref[...], x6, preferred_element_type=f32)
        b1 = c1b_ref[...]
        p0 = jnp.maximum(jnp.maximum(y[0:8], y[8:16]) + b1, 0.0)
        p1 = jnp.maximum(jnp.maximum(y[16:24], y[24:32]) + b1, 0.0)

        p1m = jnp.where(first, 0.0, jnp.roll(p1, 1, axis=1))
        p0p = jnp.where(last, 0.0, jnp.roll(p0, -1, axis=1))
        ps = jnp.concatenate([p1m, p0, p1, p0p], axis=0)
        q = jnp.dot(c2_ref[...], ps, preferred_element_type=f32)
        h2 = jnp.maximum(
            jnp.maximum(q[0:8], q[8:16]) + c2b_ref[...], 0.0)

        g1 = jnp.dot(g1w_ref[...], h2, preferred_element_type=f32)
        a1 = jnp.maximum(dinv * agg(g1 * dinv) + g1b_ref[...], 0.0)
        g2 = jnp.dot(g2w_ref[...], a1, preferred_element_type=f32)
        a2 = jnp.maximum(dinv * agg(g2 * dinv) + g2b_ref[...], 0.0)

        cols = [jnp.sum(a2[:, i * _P:(i + 1) * _P], axis=1, keepdims=True)
                for i in range(_GH)]
        return jnp.concatenate(cols, axis=1)

    xall = x_ref[...]
    pa = half(xall[:, :_GPH])
    pb = half(xall[:, _GPH:])
    pooled = jnp.concatenate([pa, pb], axis=1) * (1.0 / _P)
    out = jnp.dot(jnp.transpose(pooled), fcw_ref[...],
                  preferred_element_type=f32) + fcb_ref[...]
    out_ref[...] = out


def kernel(x, batch_size, conv1_w, conv1_b, conv2_w, conv2_b,
           gcn1_w, gcn1_b, gcn2_w, gcn2_b, fc_w, fc_b):
    b = x.shape[0]
    f32 = jnp.float32
    xp = (x.reshape(b, _P, 4).transpose(2, 0, 1).reshape(4, b * _P)
          .astype(f32))

    c1w = conv1_w.reshape(_F, 3).astype(f32)
    c1 = jnp.concatenate(
        [jnp.concatenate(
            [jnp.zeros((_F, r), f32), c1w, jnp.zeros((_F, 3 - r), f32)],
            axis=1) for r in range(4)],
        axis=0)
    w0, w1, w2 = (conv2_w[:, :, 0].astype(f32), conv2_w[:, :, 1].astype(f32),
                  conv2_w[:, :, 2].astype(f32))
    z8 = jnp.zeros((_F, _F), f32)
    c2 = jnp.concatenate(
        [jnp.concatenate([w0, w1, w2, z8], axis=1),
         jnp.concatenate([z8, w0, w1, w2], axis=1)],
        axis=0)

    args = (
        xp,
        c1,
        conv1_b.reshape(_F, 1).astype(f32),
        c2,
        conv2_b.reshape(_F, 1).astype(f32),
        gcn1_w.T.astype(f32),
        gcn1_b.reshape(_H, 1).astype(f32),
        gcn2_w.T.astype(f32),
        gcn2_b.reshape(_H, 1).astype(f32),
        fc_w.astype(f32),
        fc_b.reshape(1, _NC).astype(f32),
    )
    grid = (b * _P // _GP,)
    full = lambda shape: pl.BlockSpec(shape, lambda i: tuple(0 for _ in shape))
    out = pl.pallas_call(
        _model_block,
        grid=grid,
        in_specs=[
            pl.BlockSpec((4, _GP), lambda i: (0, i)),
            full((4 * _F, 6)),
            full((_F, 1)),
            full((2 * _F, 4 * _F)),
            full((_F, 1)),
            full((_H, _F)),
            full((_H, 1)),
            full((_H, _H)),
            full((_H, 1)),
            full((_H, _NC)),
            full((1, _NC)),
        ],
        out_specs=pl.BlockSpec((_G, _NC), lambda i: (i, 0)),
        out_shape=jax.ShapeDtypeStruct((b, _NC), f32),
        compiler_params=pltpu.CompilerParams(
            dimension_semantics=("parallel",),
        ),
    )(*args)
    return out


# speedup vs baseline: 1.2175x; 1.2175x over previous
import jax
import jax.numpy as jnp
from jax.experimental import pallas as pl
from jax.experimental.pallas import tpu as pltpu

_L = 4096
_F = 8
_H = 64
_NC = 2
_P = _L // 4
_G = 32
_NCHAIN = 8
_GH = _G // _NCHAIN
_GP = _G * _P
_GPH = _GH * _P

_ISQRT2 = 0.7071067811865476
_ISQRT3 = 0.5773502691896258


def _model_block(x_ref, c1_ref, c1b_ref, c2_ref, c2b_ref,
                 g1w_ref, g1b_ref, g2w_ref, g2b_ref,
                 fcw_ref, fcb_ref, out_ref):
    f32 = jnp.float32
    pos = jax.lax.broadcasted_iota(jnp.int32, (1, _GPH), 1) % _P
    first = pos == 0
    last = pos == _P - 1
    dinv = jnp.where(first | last, _ISQRT2, _ISQRT3).astype(f32)

    def agg(u):
        ul = jnp.where(first, 0.0, jnp.roll(u, 1, axis=1))
        ur = jnp.where(last, 0.0, jnp.roll(u, -1, axis=1))
        return ul + u + ur

    def stage_conv1(xp):
        x3m = jnp.where(first, 0.0, jnp.roll(xp[3:4], 1, axis=1))
        x0p = jnp.where(last, 0.0, jnp.roll(xp[0:1], -1, axis=1))
        x6 = jnp.concatenate([x3m, xp, x0p], axis=0)
        y = jnp.dot(c1_ref[...], x6, preferred_element_type=f32)
        b1 = c1b_ref[...]
        p0 = jnp.maximum(jnp.maximum(y[0:8], y[8:16]) + b1, 0.0)
        p1 = jnp.maximum(jnp.maximum(y[16:24], y[24:32]) + b1, 0.0)
        return p0, p1

    def stage_conv2(p0, p1):
        p1m = jnp.where(first, 0.0, jnp.roll(p1, 1, axis=1))
        p0p = jnp.where(last, 0.0, jnp.roll(p0, -1, axis=1))
        ps = jnp.concatenate([p1m, p0, p1, p0p], axis=0)
        q = jnp.dot(c2_ref[...], ps, preferred_element_type=f32)
        return jnp.maximum(
            jnp.maximum(q[0:8], q[8:16]) + c2b_ref[...], 0.0)

    def stage_gcn(wref, bref, h):
        g = jnp.dot(wref[...], h, preferred_element_type=f32)
        return jnp.maximum(dinv * agg(g * dinv) + bref[...], 0.0)

    def stage_pool(a2):
        cols = [jnp.sum(a2[:, i * _P:(i + 1) * _P], axis=1, keepdims=True)
                for i in range(_GH)]
        return jnp.concatenate(cols, axis=1)

    xall = x_ref[...]
    xs = [xall[:, k * _GPH:(k + 1) * _GPH] for k in range(_NCHAIN)]
    pps = [stage_conv1(xp) for xp in xs]
    h2s = [stage_conv2(p0, p1) for (p0, p1) in pps]
    a1s = [stage_gcn(g1w_ref, g1b_ref, h2) for h2 in h2s]
    a2s = [stage_gcn(g2w_ref, g2b_ref, a1) for a1 in a1s]
    parts = [stage_pool(a2) for a2 in a2s]
    pooled = jnp.concatenate(parts, axis=1) * (1.0 / _P)
    out = jnp.dot(jnp.transpose(pooled), fcw_ref[...],
                  preferred_element_type=f32) + fcb_ref[...]
    out_ref[...] = out


def kernel(x, batch_size, conv1_w, conv1_b, conv2_w, conv2_b,
           gcn1_w, gcn1_b, gcn2_w, gcn2_b, fc_w, fc_b):
    b = x.shape[0]
    f32 = jnp.float32
    xp = (x.reshape(b, _P, 4).transpose(2, 0, 1).reshape(4, b * _P)
          .astype(f32))

    c1w = conv1_w.reshape(_F, 3).astype(f32)
    c1 = jnp.concatenate(
        [jnp.concatenate(
            [jnp.zeros((_F, r), f32), c1w, jnp.zeros((_F, 3 - r), f32)],
            axis=1) for r in range(4)],
        axis=0)
    w0, w1, w2 = (conv2_w[:, :, 0].astype(f32), conv2_w[:, :, 1].astype(f32),
                  conv2_w[:, :, 2].astype(f32))
    z8 = jnp.zeros((_F, _F), f32)
    c2 = jnp.concatenate(
        [jnp.concatenate([w0, w1, w2, z8], axis=1),
         jnp.concatenate([z8, w0, w1, w2], axis=1)],
        axis=0)

    args = (
        xp,
        c1,
        conv1_b.reshape(_F, 1).astype(f32),
        c2,
        conv2_b.reshape(_F, 1).astype(f32),
        gcn1_w.T.astype(f32),
        gcn1_b.reshape(_H, 1).astype(f32),
        gcn2_w.T.astype(f32),
        gcn2_b.reshape(_H, 1).astype(f32),
        fc_w.astype(f32),
        fc_b.reshape(1, _NC).astype(f32),
    )
    grid = (b * _P // _GP,)
    full = lambda shape: pl.BlockSpec(shape, lambda i: tuple(0 for _ in shape))
    out = pl.pallas_call(
        _model_block,
        grid=grid,
        in_specs=[
            pl.BlockSpec((4, _GP), lambda i: (0, i)),
            full((4 * _F, 6)),
            full((_F, 1)),
            full((2 * _F, 4 * _F)),
            full((_F, 1)),
            full((_H, _F)),
            full((_H, 1)),
            full((_H, _H)),
            full((_H, 1)),
            full((_H, _NC)),
            full((1, _NC)),
        ],
        out_specs=pl.BlockSpec((_G, _NC), lambda i: (i, 0)),
        out_shape=jax.ShapeDtypeStruct((b, _NC), f32),
        compiler_params=pltpu.CompilerParams(
            dimension_semantics=("parallel",),
        ),
    )(*args)
    return out


# speedup vs baseline: 1.2542x; 1.0302x over previous
import jax
import jax.numpy as jnp
from jax.experimental import pallas as pl
from jax.experimental.pallas import tpu as pltpu

_L = 4096
_F = 8
_H = 64
_NC = 2
_P = _L // 4
_G = 32
_NCHAIN = 8
_GH = _G // _NCHAIN
_GP = _G * _P
_GPH = _GH * _P

_ISQRT2 = 0.7071067811865476
_ISQRT3 = 0.5773502691896258


def _model_block(x_ref, c1_ref, c1b_ref, c2_ref, c2b_ref,
                 g1w_ref, g1b_ref, g2w_ref, g2b_ref,
                 fcw_ref, fcb_ref, out_ref):
    f32 = jnp.float32
    pos = jax.lax.broadcasted_iota(jnp.int32, (1, _GPH), 1) % _P
    first = pos == 0
    last = pos == _P - 1
    dinv = jnp.where(first | last, _ISQRT2, _ISQRT3).astype(f32)
    cA = jnp.where(first, 0.0, dinv * jnp.roll(dinv, 1, axis=1))
    cB = dinv * dinv
    cC = jnp.where(last, 0.0, dinv * jnp.roll(dinv, -1, axis=1))

    def stage_conv1(xp):
        x3m = jnp.where(first, 0.0, jnp.roll(xp[3:4], 1, axis=1))
        x0p = jnp.where(last, 0.0, jnp.roll(xp[0:1], -1, axis=1))
        x6 = jnp.concatenate([x3m, xp, x0p], axis=0)
        y = jnp.dot(c1_ref[...], x6, preferred_element_type=f32)
        b1 = c1b_ref[...]
        p0 = jnp.maximum(jnp.maximum(y[0:8], y[8:16]) + b1, 0.0)
        p1 = jnp.maximum(jnp.maximum(y[16:24], y[24:32]) + b1, 0.0)
        return p0, p1

    def stage_conv2(p0, p1):
        p1m = jnp.where(first, 0.0, jnp.roll(p1, 1, axis=1))
        p0p = jnp.where(last, 0.0, jnp.roll(p0, -1, axis=1))
        ps = jnp.concatenate([p1m, p0, p1, p0p], axis=0)
        q = jnp.dot(c2_ref[...], ps, preferred_element_type=f32)
        return jnp.maximum(
            jnp.maximum(q[0:8], q[8:16]) + c2b_ref[...], 0.0)

    def stage_gcn(wref, bref, h):
        g = jnp.dot(wref[...], h, preferred_element_type=f32)
        s = (cA * jnp.roll(g, 1, axis=1) + cB * g
             + cC * jnp.roll(g, -1, axis=1))
        return jnp.maximum(s + bref[...], 0.0)

    def stage_pool(a2):
        cols = [jnp.sum(a2[:, i * _P:(i + 1) * _P], axis=1, keepdims=True)
                for i in range(_GH)]
        return jnp.concatenate(cols, axis=1)

    xall = x_ref[...]
    xs = [xall[:, k * _GPH:(k + 1) * _GPH] for k in range(_NCHAIN)]
    pps = [stage_conv1(xp) for xp in xs]
    h2s = [stage_conv2(p0, p1) for (p0, p1) in pps]
    a1s = [stage_gcn(g1w_ref, g1b_ref, h2) for h2 in h2s]
    a2s = [stage_gcn(g2w_ref, g2b_ref, a1) for a1 in a1s]
    parts = [stage_pool(a2) for a2 in a2s]
    pooled = jnp.concatenate(parts, axis=1) * (1.0 / _P)
    out = jnp.dot(jnp.transpose(pooled), fcw_ref[...],
                  preferred_element_type=f32) + fcb_ref[...]
    out_ref[...] = out


def kernel(x, batch_size, conv1_w, conv1_b, conv2_w, conv2_b,
           gcn1_w, gcn1_b, gcn2_w, gcn2_b, fc_w, fc_b):
    b = x.shape[0]
    f32 = jnp.float32
    xp = (x.reshape(b, _P, 4).transpose(2, 0, 1).reshape(4, b * _P)
          .astype(f32))

    c1w = conv1_w.reshape(_F, 3).astype(f32)
    c1 = jnp.concatenate(
        [jnp.concatenate(
            [jnp.zeros((_F, r), f32), c1w, jnp.zeros((_F, 3 - r), f32)],
            axis=1) for r in range(4)],
        axis=0)
    w0, w1, w2 = (conv2_w[:, :, 0].astype(f32), conv2_w[:, :, 1].astype(f32),
                  conv2_w[:, :, 2].astype(f32))
    z8 = jnp.zeros((_F, _F), f32)
    c2 = jnp.concatenate(
        [jnp.concatenate([w0, w1, w2, z8], axis=1),
         jnp.concatenate([z8, w0, w1, w2], axis=1)],
        axis=0)

    args = (
        xp,
        c1,
        conv1_b.reshape(_F, 1).astype(f32),
        c2,
        conv2_b.reshape(_F, 1).astype(f32),
        gcn1_w.T.astype(f32),
        gcn1_b.reshape(_H, 1).astype(f32),
        gcn2_w.T.astype(f32),
        gcn2_b.reshape(_H, 1).astype(f32),
        fc_w.astype(f32),
        fc_b.reshape(1, _NC).astype(f32),
    )
    grid = (b * _P // _GP,)
    full = lambda shape: pl.BlockSpec(shape, lambda i: tuple(0 for _ in shape))
    out = pl.pallas_call(
        _model_block,
        grid=grid,
        in_specs=[
            pl.BlockSpec((4, _GP), lambda i: (0, i)),
            full((4 * _F, 6)),
            full((_F, 1)),
            full((2 * _F, 4 * _F)),
            full((_F, 1)),
            full((_H, _F)),
            full((_H, 1)),
            full((_H, _H)),
            full((_H, 1)),
            full((_H, _NC)),
            full((1, _NC)),
        ],
        out_specs=pl.BlockSpec((_G, _NC), lambda i: (i, 0)),
        out_shape=jax.ShapeDtypeStruct((b, _NC), f32),
        compiler_params=pltpu.CompilerParams(
            dimension_semantics=("parallel",),
        ),
    )(*args)
    return out


# speedup vs baseline: 1.2906x; 1.0290x over previous
import jax
import jax.numpy as jnp
from jax.experimental import pallas as pl
from jax.experimental.pallas import tpu as pltpu

_L = 4096
_F = 8
_H = 64
_NC = 2
_P = _L // 4
_G = 32
_NCHAIN = 16
_GH = _G // _NCHAIN
_GP = _G * _P
_GPH = _GH * _P

_ISQRT2 = 0.7071067811865476
_ISQRT3 = 0.5773502691896258


def _model_block(x_ref, c1_ref, c1b_ref, c2_ref, c2b_ref,
                 g1w_ref, g1b_ref, g2w_ref, g2b_ref,
                 fcw_ref, fcb_ref, out_ref):
    f32 = jnp.float32
    pos = jax.lax.broadcasted_iota(jnp.int32, (1, _GPH), 1) % _P
    first = pos == 0
    last = pos == _P - 1
    dinv = jnp.where(first | last, _ISQRT2, _ISQRT3).astype(f32)
    cA = jnp.where(first, 0.0, dinv * jnp.roll(dinv, 1, axis=1))
    cB = dinv * dinv
    cC = jnp.where(last, 0.0, dinv * jnp.roll(dinv, -1, axis=1))

    def stage_conv1(xp):
        x3m = jnp.where(first, 0.0, jnp.roll(xp[3:4], 1, axis=1))
        x0p = jnp.where(last, 0.0, jnp.roll(xp[0:1], -1, axis=1))
        x6 = jnp.concatenate([x3m, xp, x0p], axis=0)
        y = jnp.dot(c1_ref[...], x6, preferred_element_type=f32)
        b1 = c1b_ref[...]
        p0 = jnp.maximum(jnp.maximum(y[0:8], y[8:16]) + b1, 0.0)
        p1 = jnp.maximum(jnp.maximum(y[16:24], y[24:32]) + b1, 0.0)
        return p0, p1

    def stage_conv2(p0, p1):
        p1m = jnp.where(first, 0.0, jnp.roll(p1, 1, axis=1))
        p0p = jnp.where(last, 0.0, jnp.roll(p0, -1, axis=1))
        ps = jnp.concatenate([p1m, p0, p1, p0p], axis=0)
        q = jnp.dot(c2_ref[...], ps, preferred_element_type=f32)
        return jnp.maximum(
            jnp.maximum(q[0:8], q[8:16]) + c2b_ref[...], 0.0)

    def stage_gcn(wref, bref, h):
        g = jnp.dot(wref[...], h, preferred_element_type=f32)
        s = (cA * jnp.roll(g, 1, axis=1) + cB * g
             + cC * jnp.roll(g, -1, axis=1))
        return jnp.maximum(s + bref[...], 0.0)

    def stage_pool(a2):
        cols = [jnp.sum(a2[:, i * _P:(i + 1) * _P], axis=1, keepdims=True)
                for i in range(_GH)]
        return jnp.concatenate(cols, axis=1)

    xall = x_ref[...]
    xs = [xall[:, k * _GPH:(k + 1) * _GPH] for k in range(_NCHAIN)]
    pps = [stage_conv1(xp) for xp in xs]
    h2s = [stage_conv2(p0, p1) for (p0, p1) in pps]
    a1s = [stage_gcn(g1w_ref, g1b_ref, h2) for h2 in h2s]
    a2s = [stage_gcn(g2w_ref, g2b_ref, a1) for a1 in a1s]
    parts = [stage_pool(a2) for a2 in a2s]
    pooled = jnp.concatenate(parts, axis=1) * (1.0 / _P)
    out = jnp.dot(jnp.transpose(pooled), fcw_ref[...],
                  preferred_element_type=f32) + fcb_ref[...]
    out_ref[...] = out


def kernel(x, batch_size, conv1_w, conv1_b, conv2_w, conv2_b,
           gcn1_w, gcn1_b, gcn2_w, gcn2_b, fc_w, fc_b):
    b = x.shape[0]
    f32 = jnp.float32
    xp = (x.reshape(b, _P, 4).transpose(2, 0, 1).reshape(4, b * _P)
          .astype(f32))

    c1w = conv1_w.reshape(_F, 3).astype(f32)
    c1 = jnp.concatenate(
        [jnp.concatenate(
            [jnp.zeros((_F, r), f32), c1w, jnp.zeros((_F, 3 - r), f32)],
            axis=1) for r in range(4)],
        axis=0)
    w0, w1, w2 = (conv2_w[:, :, 0].astype(f32), conv2_w[:, :, 1].astype(f32),
                  conv2_w[:, :, 2].astype(f32))
    z8 = jnp.zeros((_F, _F), f32)
    c2 = jnp.concatenate(
        [jnp.concatenate([w0, w1, w2, z8], axis=1),
         jnp.concatenate([z8, w0, w1, w2], axis=1)],
        axis=0)

    args = (
        xp,
        c1,
        conv1_b.reshape(_F, 1).astype(f32),
        c2,
        conv2_b.reshape(_F, 1).astype(f32),
        gcn1_w.T.astype(f32),
        gcn1_b.reshape(_H, 1).astype(f32),
        gcn2_w.T.astype(f32),
        gcn2_b.reshape(_H, 1).astype(f32),
        fc_w.astype(f32),
        fc_b.reshape(1, _NC).astype(f32),
    )
    grid = (b * _P // _GP,)
    full = lambda shape: pl.BlockSpec(shape, lambda i: tuple(0 for _ in shape))
    out = pl.pallas_call(
        _model_block,
        grid=grid,
        in_specs=[
            pl.BlockSpec((4, _GP), lambda i: (0, i)),
            full((4 * _F, 6)),
            full((_F, 1)),
            full((2 * _F, 4 * _F)),
            full((_F, 1)),
            full((_H, _F)),
            full((_H, 1)),
            full((_H, _H)),
            full((_H, 1)),
            full((_H, _NC)),
            full((1, _NC)),
        ],
        out_specs=pl.BlockSpec((_G, _NC), lambda i: (i, 0)),
        out_shape=jax.ShapeDtypeStruct((b, _NC), f32),
        compiler_params=pltpu.CompilerParams(
            dimension_semantics=("parallel",),
        ),
    )(*args)
    return out
